# baseline (speedup 1.0000x reference)
import functools

import jax
import jax.numpy as jnp
from jax import lax
from jax.experimental import pallas as pl
from jax.experimental.pallas import tpu as pltpu
from jax.experimental.pallas import tpu_sc as plsc

L = 16
KG = 8
SR = 8


@functools.lru_cache(maxsize=None)
def _build_permute(stripes: int, hw: int, chunk_stripes: int):
    info = plsc.get_sparse_core_info()
    nc, ns = info.num_cores, info.num_subcores
    nw = nc * ns
    assert stripes % (nw * chunk_stripes) == 0
    spw = stripes // nw
    nchunk = spw // chunk_stripes
    nk = hw // L
    rows = chunk_stripes * SR

    mesh = plsc.VectorSubcoreMesh(core_axis_name="c", subcore_axis_name="s")

    @functools.partial(
        pl.kernel,
        mesh=mesh,
        out_type=jax.ShapeDtypeStruct((stripes, SR, hw), jnp.float32),
        scratch_types=[
            pltpu.VMEM((hw,), jnp.int32),
            pltpu.VMEM((2, chunk_stripes, SR, hw), jnp.float32),
            pltpu.VMEM((2, chunk_stripes, SR, hw), jnp.float32),
            pltpu.SemaphoreType.DMA,
            pltpu.SemaphoreType.DMA,
            pltpu.SemaphoreType.DMA,
            pltpu.SemaphoreType.DMA,
        ],
        compiler_params=pltpu.CompilerParams(
            needs_layout_passes=False, use_tc_tiling_on_sc=True
        ),
    )
    def permute(x_hbm, idx_hbm, out_hbm, idx_v, in_v, out_v,
                sin0, sin1, sout0, sout1):
        wid = lax.axis_index("s") * nc + lax.axis_index("c")
        base = wid * spw
        sins = (sin0, sin1)
        souts = (sout0, sout1)
        pltpu.sync_copy(idx_hbm, idx_v)

        def in_copy(ci, b):
            s0 = base + ci * chunk_stripes
            return pltpu.make_async_copy(
                x_hbm.at[pl.ds(s0, chunk_stripes)], in_v.at[b], sins[b]
            )

        def out_copy(ci, b):
            s0 = base + ci * chunk_stripes
            return pltpu.make_async_copy(
                out_v.at[b], out_hbm.at[pl.ds(s0, chunk_stripes)], souts[b]
            )

        def compute(b):
            def kbody(kb, c2):
                k0 = kb * KG
                idxs = [idx_v[pl.ds((k0 + j) * L, L)] for j in range(KG)]

                @plsc.parallel_loop(0, rows, unroll=2)
                def rbody(r):
                    s = r // SR
                    r8 = r % SR
                    sv = jnp.full((L,), s, jnp.int32)
                    rv = jnp.full((L,), r8, jnp.int32)
                    for j in range(KG):
                        vals = plsc.load_gather(in_v.at[b], [sv, rv, idxs[j]])
                        out_v[b, s, r8, pl.ds((k0 + j) * L, L)] = vals

                return c2

            lax.fori_loop(0, nk // KG, kbody, 0)

        in_copy(0, 0).start()

        def pipe_body(i2, carry):
            for ph in range(2):
                ci = i2 * 2 + ph
                in_copy(ci, ph).wait()

                @pl.when(ci + 1 < nchunk)
                def _():
                    in_copy(ci + 1, 1 - ph).start()

                @pl.when(ci >= 2)
                def _():
                    out_copy(ci - 2, ph).wait()

                compute(ph)
                out_copy(ci, ph).start()
            return carry

        lax.fori_loop(0, nchunk // 2, pipe_body, 0)
        out_copy(nchunk - 2, 0).wait()
        out_copy(nchunk - 1, 1).wait()

    return permute


def kernel(x, rd_index):
    b, c, hw = x.shape
    stripes = b * c // SR
    permute = _build_permute(stripes, hw, 2)
    out = permute(x.reshape(stripes, SR, hw), rd_index)
    return out.reshape(b, c, hw)


# speedup vs baseline: 1.0388x; 1.0388x over previous
import functools

import jax
import jax.numpy as jnp
from jax import lax
from jax.experimental import pallas as pl
from jax.experimental.pallas import tpu as pltpu
from jax.experimental.pallas import tpu_sc as plsc

L = 16
KG = 8
SR = 8


@functools.lru_cache(maxsize=None)
def _build_permute(stripes: int, hw: int, chunk_stripes: int):
    info = plsc.get_sparse_core_info()
    nc, ns = info.num_cores, info.num_subcores
    nw = nc * ns
    assert stripes % (nw * chunk_stripes) == 0
    spw = stripes // nw
    nchunk = spw // chunk_stripes
    nk = hw // L
    rows = chunk_stripes * SR

    mesh = plsc.VectorSubcoreMesh(core_axis_name="c", subcore_axis_name="s")

    @functools.partial(
        pl.kernel,
        mesh=mesh,
        out_type=jax.ShapeDtypeStruct((stripes, SR, hw), jnp.float32),
        scratch_types=[
            pltpu.VMEM((hw,), jnp.int32),
            pltpu.VMEM((2, chunk_stripes, SR, hw), jnp.float32),
            pltpu.VMEM((2, chunk_stripes, SR, hw), jnp.float32),
            pltpu.SemaphoreType.DMA,
            pltpu.SemaphoreType.DMA,
            pltpu.SemaphoreType.DMA,
            pltpu.SemaphoreType.DMA,
        ],
        compiler_params=pltpu.CompilerParams(
            needs_layout_passes=False, use_tc_tiling_on_sc=True
        ),
    )
    def permute(x_hbm, idx_hbm, out_hbm, idx_v, in_v, out_v,
                sin0, sin1, sout0, sout1):
        wid = lax.axis_index("s") * nc + lax.axis_index("c")
        base = wid * spw
        sins = (sin0, sin1)
        souts = (sout0, sout1)
        pltpu.sync_copy(idx_hbm, idx_v)

        class _Multi:
            def __init__(self, copies):
                self.copies = copies

            def start(self):
                for c in self.copies:
                    c.start()

            def wait(self):
                for c in self.copies:
                    c.wait()

        def in_copy(ci, b):
            s0 = base + ci * chunk_stripes
            return _Multi([
                pltpu.make_async_copy(
                    x_hbm.at[pl.ds(s0 + j, 1)], in_v.at[b, pl.ds(j, 1)],
                    sins[b],
                )
                for j in range(chunk_stripes)
            ])

        def out_copy(ci, b):
            s0 = base + ci * chunk_stripes
            return _Multi([
                pltpu.make_async_copy(
                    out_v.at[b, pl.ds(j, 1)], out_hbm.at[pl.ds(s0 + j, 1)],
                    souts[b],
                )
                for j in range(chunk_stripes)
            ])

        def compute(b):
            def kbody(kb, c2):
                k0 = kb * KG
                idxs = [idx_v[pl.ds((k0 + j) * L, L)] for j in range(KG)]

                @plsc.parallel_loop(0, rows, unroll=2)
                def rbody(r):
                    s = r // SR
                    r8 = r % SR
                    sv = jnp.full((L,), s, jnp.int32)
                    rv = jnp.full((L,), r8, jnp.int32)
                    for j in range(KG):
                        vals = plsc.load_gather(in_v.at[b], [sv, rv, idxs[j]])
                        out_v[b, s, r8, pl.ds((k0 + j) * L, L)] = vals

                return c2

            lax.fori_loop(0, nk // KG, kbody, 0)

        in_copy(0, 0).start()

        def pipe_body(i2, carry):
            for ph in range(2):
                ci = i2 * 2 + ph
                in_copy(ci, ph).wait()

                @pl.when(ci + 1 < nchunk)
                def _():
                    in_copy(ci + 1, 1 - ph).start()

                @pl.when(ci >= 2)
                def _():
                    out_copy(ci - 2, ph).wait()

                compute(ph)
                out_copy(ci, ph).start()
            return carry

        lax.fori_loop(0, nchunk // 2, pipe_body, 0)
        out_copy(nchunk - 2, 0).wait()
        out_copy(nchunk - 1, 1).wait()

    return permute


def kernel(x, rd_index):
    b, c, hw = x.shape
    stripes = b * c // SR
    permute = _build_permute(stripes, hw, 3)
    out = permute(x.reshape(stripes, SR, hw), rd_index)
    return out.reshape(b, c, hw)


# speedup vs baseline: 1.0505x; 1.0113x over previous
import functools

import jax
import jax.numpy as jnp
from jax import lax
from jax.experimental import pallas as pl
from jax.experimental.pallas import tpu as pltpu
from jax.experimental.pallas import tpu_sc as plsc

L = 16
KG = 8
SR = 8


@functools.lru_cache(maxsize=None)
def _build_permute(stripes: int, hw: int, chunk_stripes: int):
    info = plsc.get_sparse_core_info()
    nc, ns = info.num_cores, info.num_subcores
    nw = nc * ns
    assert stripes % (nw * chunk_stripes) == 0
    spw = stripes // nw
    nchunk = spw // chunk_stripes
    assert nchunk % 2 == 0
    nk = hw // L
    rows = chunk_stripes * SR

    mesh = plsc.VectorSubcoreMesh(core_axis_name="c", subcore_axis_name="s")

    @functools.partial(
        pl.kernel,
        mesh=mesh,
        out_type=jax.ShapeDtypeStruct((stripes, SR, hw), jnp.float32),
        scratch_types=[
            pltpu.VMEM((hw,), jnp.int32),
            pltpu.VMEM((2, chunk_stripes, SR, hw), jnp.float32),
            pltpu.VMEM((2, chunk_stripes, SR, hw), jnp.float32),
            pltpu.SemaphoreType.DMA,
            pltpu.SemaphoreType.DMA,
            pltpu.SemaphoreType.DMA,
            pltpu.SemaphoreType.DMA,
        ],
        compiler_params=pltpu.CompilerParams(
            needs_layout_passes=False, use_tc_tiling_on_sc=True
        ),
    )
    def permute(x_hbm, idx_hbm, out_hbm, idx_v, in_v, out_v,
                sin0, sin1, sout0, sout1):
        wid = lax.axis_index("s") * nc + lax.axis_index("c")
        base = wid * spw
        sins = (sin0, sin1)
        souts = (sout0, sout1)
        pltpu.sync_copy(idx_hbm, idx_v)

        def in_copy(ci, b):
            s0 = base + ci * chunk_stripes
            return pltpu.make_async_copy(
                x_hbm.at[pl.ds(s0, chunk_stripes)], in_v.at[b], sins[b]
            )

        def out_copy(ci, b):
            s0 = base + ci * chunk_stripes
            return pltpu.make_async_copy(
                out_v.at[b], out_hbm.at[pl.ds(s0, chunk_stripes)], souts[b]
            )

        def compute(b):
            def kbody(kb, c2):
                k0 = kb * KG
                idxs = [idx_v[pl.ds((k0 + j) * L, L)] for j in range(KG)]

                @plsc.parallel_loop(0, rows, unroll=2)
                def rbody(r):
                    s = r // SR
                    r8 = r % SR
                    sv = jnp.full((L,), s, jnp.int32)
                    rv = jnp.full((L,), r8, jnp.int32)
                    for j in range(KG):
                        vals = plsc.load_gather(in_v.at[b], [sv, rv, idxs[j]])
                        out_v[b, s, r8, pl.ds((k0 + j) * L, L)] = vals

                return c2

            lax.fori_loop(0, nk // KG, kbody, 0)

        in_copy(0, 0).start()

        def pipe_body(i2, carry):
            for ph in range(2):
                ci = i2 * 2 + ph
                in_copy(ci, ph).wait()

                @pl.when(ci + 1 < nchunk)
                def _():
                    in_copy(ci + 1, 1 - ph).start()

                @pl.when(ci >= 2)
                def _():
                    out_copy(ci - 2, ph).wait()

                compute(ph)
                out_copy(ci, ph).start()
            return carry

        lax.fori_loop(0, nchunk // 2, pipe_body, 0)
        out_copy(nchunk - 2, 0).wait()
        out_copy(nchunk - 1, 1).wait()

    return permute


def kernel(x, rd_index):
    b, c, hw = x.shape
    stripes = b * c // SR
    permute = _build_permute(stripes, hw, 3)
    out = permute(x.reshape(stripes, SR, hw), rd_index)
    return out.reshape(b, c, hw)
